# baseline (speedup 1.0000x reference)
import functools

import jax
import jax.numpy as jnp
from jax import lax
from jax.experimental import pallas as pl
from jax.experimental.pallas import tpu as pltpu
from jax.experimental.pallas import tpu_sc as plsc

N = 10000
E = 320000
D_IN = 128
H = 128
H2 = 64
G = 128
OUT = 10

NP = 10240
PAD = NP - N
NC = 2
NS = 16
LANES = 16
CH = 128
EPT = 20736
E_PAD = EPT * NS
NCHUNKT = EPT // CH
DW = 80


def _lrelu(v):
    return jnp.where(v > 0, v, 0.2 * v)


def _fold(hhalf):
    n = hhalf.shape[0]
    blk = jnp.concatenate(
        [hhalf, jnp.ones((n, 1), jnp.float32),
         jnp.zeros((n, DW - H2 - 1), jnp.float32)], axis=1)
    return jnp.concatenate([blk, jnp.zeros((PAD, DW), jnp.float32)], axis=0)


def _fold_init(num0half, wself):
    n = num0half.shape[0]
    blk = jnp.concatenate(
        [num0half, wself, jnp.zeros((n, DW - H2 - 1), jnp.float32)], axis=1)
    return jnp.concatenate([blk, jnp.zeros((PAD, DW), jnp.float32)], axis=0)


def _pad_col(v):
    return jnp.concatenate([v, jnp.zeros((PAD, 1), jnp.float32)], axis=0)


def _prep1_body(x_ref, w_ref, as_ref, ad_ref,
                h_out, inum_out, as_out, ad_out, maxs_out):
    h = jnp.dot(x_ref[...], w_ref[...], preferred_element_type=jnp.float32)
    a_s = jnp.dot(h, as_ref[...].reshape(H, 1))
    a_d = jnp.dot(h, ad_ref[...].reshape(H, 1))
    maxs = jnp.max(a_s)
    c = _lrelu(maxs + a_d)
    wself = jnp.exp(_lrelu(a_s + a_d) - c)
    num0 = wself * h
    h_out[0] = _fold(h[:, :H // 2])
    h_out[1] = _fold(h[:, H // 2:])
    inum_out[0] = _fold_init(num0[:, :H // 2], wself)
    inum_out[1] = _fold_init(num0[:, H // 2:], wself)
    asp = _pad_col(a_s)
    as_out[0] = asp
    as_out[1] = asp
    ad_out[...] = _pad_col(a_d)
    maxs_out[...] = jnp.full((1, LANES), maxs, jnp.float32)


_prep1 = pl.pallas_call(
    _prep1_body,
    compiler_params=pltpu.CompilerParams(vmem_limit_bytes=100 * 1024 * 1024),
    out_shape=(
        jax.ShapeDtypeStruct((2, NP, DW), jnp.float32),
        jax.ShapeDtypeStruct((2, NP, DW), jnp.float32),
        jax.ShapeDtypeStruct((2, NP, 1), jnp.float32),
        jax.ShapeDtypeStruct((NP, 1), jnp.float32),
        jax.ShapeDtypeStruct((1, LANES), jnp.float32),
    ),
)


def _fin1_prep2_body(numa_ref, numb_ref, b1_ref, g1_ref, be1_ref,
                     w2_ref, as2_ref, ad2_ref,
                     h_out, inum_out, as_out, ad_out, maxs_out):
    num = jnp.concatenate([numa_ref[...][:N, :H // 2],
                           numb_ref[...][:N, :H // 2]], axis=1)
    den = numa_ref[...][:N, H // 2:H // 2 + 1]
    o = num / (den + 1e-16) + b1_ref[...].reshape(1, H)
    mu = jnp.mean(o, axis=0, keepdims=True)
    var = jnp.mean((o - mu) * (o - mu), axis=0, keepdims=True)
    o = (o - mu) / jnp.sqrt(var + 1e-5) * g1_ref[...].reshape(1, H) \
        + be1_ref[...].reshape(1, H)
    o = jnp.maximum(o, 0.0)
    h2 = jnp.dot(o, w2_ref[...], preferred_element_type=jnp.float32)
    a_s = jnp.dot(h2, as2_ref[...].reshape(H2, 1))
    a_d = jnp.dot(h2, ad2_ref[...].reshape(H2, 1))
    maxs = jnp.max(a_s)
    c = _lrelu(maxs + a_d)
    wself = jnp.exp(_lrelu(a_s + a_d) - c)
    h2f = _fold(h2)
    h_out[0] = h2f
    h_out[1] = h2f
    num0f = _fold_init(wself * h2, wself)
    inum_out[0] = num0f
    inum_out[1] = jnp.zeros((NP, DW), jnp.float32)
    asp = _pad_col(a_s)
    as_out[0] = asp
    as_out[1] = asp
    ad_out[...] = _pad_col(a_d)
    maxs_out[...] = jnp.full((1, LANES), maxs, jnp.float32)


_fin1_prep2 = pl.pallas_call(
    _fin1_prep2_body,
    compiler_params=pltpu.CompilerParams(vmem_limit_bytes=100 * 1024 * 1024),
    out_shape=(
        jax.ShapeDtypeStruct((2, NP, DW), jnp.float32),
        jax.ShapeDtypeStruct((2, NP, DW), jnp.float32),
        jax.ShapeDtypeStruct((2, NP, 1), jnp.float32),
        jax.ShapeDtypeStruct((NP, 1), jnp.float32),
        jax.ShapeDtypeStruct((1, LANES), jnp.float32),
    ),
)


def _fin2_body(numa_ref, numb_ref, b2_ref, g2_ref, be2_ref, batch_ref,
               wfc_ref, bfc_ref, out_ref):
    acc = numa_ref[...][:N] + numb_ref[...][:N]
    num = acc[:, :H2]
    den = acc[:, H2:H2 + 1]
    o = num / (den + 1e-16) + b2_ref[...].reshape(1, H2)
    mu = jnp.mean(o, axis=0, keepdims=True)
    var = jnp.mean((o - mu) * (o - mu), axis=0, keepdims=True)
    o = (o - mu) / jnp.sqrt(var + 1e-5) * g2_ref[...].reshape(1, H2) \
        + be2_ref[...].reshape(1, H2)
    o = jnp.maximum(o, 0.0)
    grp = lax.broadcasted_iota(jnp.int32, (N, G), 1)
    P = (batch_ref[...] == grp).astype(jnp.float32)
    cnum = ((0,), (0,)), ((), ())
    pooled = lax.dot_general(P, o, dimension_numbers=cnum,
                             preferred_element_type=jnp.float32)
    counts = lax.dot_general(P, jnp.ones((N, 1), jnp.float32),
                             dimension_numbers=cnum,
                             preferred_element_type=jnp.float32)
    pooled = pooled / jnp.maximum(counts, 1.0)
    out_ref[...] = jnp.dot(pooled, wfc_ref[...],
                           preferred_element_type=jnp.float32) \
        + bfc_ref[...].reshape(1, OUT)


_fin2 = pl.pallas_call(
    _fin2_body,
    compiler_params=pltpu.CompilerParams(vmem_limit_bytes=100 * 1024 * 1024),
    out_shape=jax.ShapeDtypeStruct((G, OUT), jnp.float32),
)


@functools.lru_cache(maxsize=None)
def _make_edge_kernel(edge_split):
    nchunk = NCHUNKT // 2 if edge_split else NCHUNKT
    NSP = 3
    spchunk = nchunk // NSP
    nbody = spchunk // 3
    rpt = NP // NS

    mesh = plsc.VectorSubcoreMesh(core_axis_name="c", subcore_axis_name="s",
                                  num_cores=NC, num_subcores=NS)

    @functools.partial(
        pl.kernel,
        out_type=jax.ShapeDtypeStruct((NC, NP, DW), jnp.float32),
        mesh=mesh,
        compiler_params=pltpu.CompilerParams(use_tc_tiling_on_sc=False),
        scratch_types=dict(
            sh_num=pltpu.VMEM_SHARED((NP, DW), jnp.float32),
            sh_as=pltpu.VMEM_SHARED((2 * NP,), jnp.float32),
            sh_ad=pltpu.VMEM_SHARED((NP,), jnp.float32),
            gidx=pltpu.VMEM((spchunk, CH), jnp.int32),
            didx=pltpu.VMEM((spchunk, CH), jnp.int32),
            asb=pltpu.VMEM((spchunk, CH), jnp.float32),
            adb=pltpu.VMEM((spchunk, CH), jnp.float32),
            maxs_t=pltpu.VMEM((LANES,), jnp.float32),
            rows=pltpu.VMEM((3, CH, DW), jnp.float32),
            sem_s=pltpu.SemaphoreType.DMA,
            sem_r0=pltpu.SemaphoreType.DMA,
            sem_r1=pltpu.SemaphoreType.DMA,
            sem_r2=pltpu.SemaphoreType.DMA,
            sem_w0=pltpu.SemaphoreType.DMA,
            sem_w1=pltpu.SemaphoreType.DMA,
            sem_w2=pltpu.SemaphoreType.DMA,
        ),
    )
    def edge_kernel(srcg, dst2d, h_hbm, as_hbm, ad_hbm, maxs_hbm,
                    inum_hbm, num_out,
                    sh_num, sh_as, sh_ad, gidx, didx, asb, adb,
                    maxs_t, rows, sem_s, sem_r0, sem_r1, sem_r2,
                    sem_w0, sem_w1, sem_w2):
        cid = lax.axis_index("c")
        sid = lax.axis_index("s")
        r0 = sid * rpt
        if edge_split:
            chunk0 = (cid * NS + sid) * nchunk
        else:
            chunk0 = sid * nchunk

        pltpu.sync_copy(inum_hbm.at[cid, pl.ds(r0, rpt)],
                        sh_num.at[pl.ds(r0, rpt)])
        pltpu.sync_copy(as_hbm.at[pl.ds(r0, rpt)], sh_as.at[pl.ds(r0, rpt)])
        pltpu.sync_copy(as_hbm.at[pl.ds(NP + r0, rpt)],
                        sh_as.at[pl.ds(NP + r0, rpt)])
        pltpu.sync_copy(ad_hbm.at[pl.ds(r0, rpt)], sh_ad.at[pl.ds(r0, rpt)])
        pltpu.sync_copy(maxs_hbm, maxs_t)
        plsc.subcore_barrier()
        maxv = maxs_t[...]

        def sg_issue(g, _):
            pltpu.async_copy(sh_as.at[gidx.at[g]], asb.at[g], sem_s)
            pltpu.async_copy(sh_ad.at[didx.at[g]], adb.at[g], sem_s)
            return 0

        def sg_drain(g, _):
            pltpu.make_async_copy(sh_as.at[gidx.at[g]], asb.at[g],
                                  sem_s).wait()
            pltpu.make_async_copy(sh_ad.at[didx.at[g]], adb.at[g],
                                  sem_s).wait()
            return 0

        def w_body(g, _):
            for m in range(CH // LANES):
                sl = pl.ds(m * LANES, LANES)
                asg = asb[g, sl]
                adg = adb[g, sl]
                e = asg + adg
                e = jnp.where(e > 0, e, 0.2 * e)
                cg = maxv + adg
                cg = jnp.where(cg > 0, cg, 0.2 * cg)
                asb[g, sl] = jnp.exp(e - cg)
            return 0

        def fetch(g, buf, sem_r):
            pltpu.async_copy(h_hbm.at[gidx.at[g]], rows.at[buf], sem_r)

        def drain(buf, sem_w):
            pltpu.make_async_copy(rows.at[buf], sh_num.at[didx.at[0]],
                                  sem_w).wait()

        def compute(g, buf, sem_r, sem_w):
            pltpu.make_async_copy(h_hbm.at[gidx.at[g]], rows.at[buf],
                                  sem_r).wait()

            def scale_body(m, _):
                wv = asb[g, pl.ds(m * LANES, LANES)]
                for l in range(LANES):
                    k = m * LANES + l
                    wsc = jnp.full((LANES,), wv[l], jnp.float32)
                    for q in range(DW // LANES):
                        sl = pl.ds(q * LANES, LANES)
                        rows[buf, k, sl] = rows[buf, k, sl] * wsc
                return 0

            lax.fori_loop(0, CH // LANES, scale_body, 0)
            pltpu.async_copy(rows.at[buf], sh_num.at[didx.at[g]], sem_w,
                             add=True)

        for sp in range(NSP):
            base = chunk0 + sp * spchunk
            pltpu.sync_copy(srcg.at[cid, pl.ds(base, spchunk)], gidx)
            pltpu.sync_copy(dst2d.at[pl.ds(base, spchunk)], didx)
            lax.fori_loop(0, spchunk, sg_issue, 0)
            lax.fori_loop(0, spchunk, sg_drain, 0)
            lax.fori_loop(0, spchunk, w_body, 0)

            fetch(0, 0, sem_r0)
            fetch(1, 1, sem_r1)

            def body(t, _):
                a = 3 * t
                compute(a, 0, sem_r0, sem_w0)

                @pl.when(t > 0)
                def _():
                    drain(2, sem_w2)
                fetch(a + 2, 2, sem_r2)
                compute(a + 1, 1, sem_r1, sem_w1)
                drain(0, sem_w0)

                @pl.when(t + 1 < nbody)
                def _():
                    fetch(a + 3, 0, sem_r0)
                compute(a + 2, 2, sem_r2, sem_w2)
                drain(1, sem_w1)

                @pl.when(t + 1 < nbody)
                def _():
                    fetch(a + 4, 1, sem_r1)
                return 0

            lax.fori_loop(0, nbody, body, 0)
            drain(2, sem_w2)

        plsc.subcore_barrier()
        pltpu.sync_copy(sh_num.at[pl.ds(r0, rpt)],
                        num_out.at[cid, pl.ds(r0, rpt)])

    return edge_kernel


def kernel(x, edge_index, batch, W1, att_src1, att_dst1, b1, g1, be1,
           W2, att_src2, att_dst2, b2, g2, be2, Wfc, bfc):
    epad = jnp.full((E_PAD - E,), N, jnp.int32)
    src2d = jnp.concatenate([edge_index[0], epad]).reshape(E_PAD // CH, CH)
    dst2d = jnp.concatenate([edge_index[1], epad]).reshape(E_PAD // CH, CH)
    srcg = jnp.stack([src2d, src2d + NP])

    _edge1 = _make_edge_kernel(False)
    _edge2 = _make_edge_kernel(True)

    h1, inum1, as1, ad1, maxs1 = _prep1(x, W1, att_src1, att_dst1)
    num1 = _edge1(srcg, dst2d, h1.reshape(2 * NP, DW),
                  as1.reshape(2 * NP), ad1.reshape(NP), maxs1.reshape(LANES),
                  inum1)
    h2, inum2, as2, ad2, maxs2 = _fin1_prep2(
        num1[0], num1[1], b1, g1, be1, W2, att_src2, att_dst2)
    num2 = _edge2(srcg, dst2d, h2.reshape(2 * NP, DW),
                  as2.reshape(2 * NP), ad2.reshape(NP), maxs2.reshape(LANES),
                  inum2)
    out = _fin2(num2[0], num2[1], b2, g2, be2, batch.reshape(N, 1), Wfc, bfc)
    return out


# speedup vs baseline: 1.9079x; 1.9079x over previous
import functools

import jax
import jax.numpy as jnp
from jax import lax
from jax.experimental import pallas as pl
from jax.experimental.pallas import tpu as pltpu
from jax.experimental.pallas import tpu_sc as plsc

N = 10000
E = 320000
D_IN = 128
H = 128
H2 = 64
G = 128
OUT = 10

NP = 10240
PAD = NP - N
NC = 2
NS = 16
LANES = 16
CH = 400
SUB = 80
NSUB = CH // SUB
DW = 80


def _lrelu(v):
    return jnp.where(v > 0, v, 0.2 * v)


def _fold(hhalf):
    n = hhalf.shape[0]
    blk = jnp.concatenate(
        [hhalf, jnp.ones((n, 1), jnp.float32),
         jnp.zeros((n, DW - H2 - 1), jnp.float32)], axis=1)
    return jnp.concatenate([blk, jnp.zeros((PAD, DW), jnp.float32)], axis=0)


def _fold_init(num0half, wself):
    n = num0half.shape[0]
    blk = jnp.concatenate(
        [num0half, wself, jnp.zeros((n, DW - H2 - 1), jnp.float32)], axis=1)
    return jnp.concatenate([blk, jnp.zeros((PAD, DW), jnp.float32)], axis=0)


def _pad_col(v):
    return jnp.concatenate([v, jnp.zeros((PAD, 1), jnp.float32)], axis=0)


def _prep1_body(x_ref, w_ref, as_ref, ad_ref,
                h_out, inum_out, as_out, ad_out, maxs_out):
    h = jnp.dot(x_ref[...], w_ref[...], preferred_element_type=jnp.float32)
    a_s = jnp.dot(h, as_ref[...].reshape(H, 1))
    a_d = jnp.dot(h, ad_ref[...].reshape(H, 1))
    maxs = jnp.max(a_s)
    c = _lrelu(maxs + a_d)
    wself = jnp.exp(_lrelu(a_s + a_d) - c)
    num0 = wself * h
    h_out[0] = _fold(h[:, :H // 2])
    h_out[1] = _fold(h[:, H // 2:])
    inum_out[0] = _fold_init(num0[:, :H // 2], wself)
    inum_out[1] = _fold_init(num0[:, H // 2:], wself)
    as_out[...] = _pad_col(a_s)
    ad_out[...] = _pad_col(a_d)
    maxs_out[...] = jnp.full((1, LANES), maxs, jnp.float32)


_prep1 = pl.pallas_call(
    _prep1_body,
    out_shape=(
        jax.ShapeDtypeStruct((2, NP, DW), jnp.float32),
        jax.ShapeDtypeStruct((2, NP, DW), jnp.float32),
        jax.ShapeDtypeStruct((NP, 1), jnp.float32),
        jax.ShapeDtypeStruct((NP, 1), jnp.float32),
        jax.ShapeDtypeStruct((1, LANES), jnp.float32),
    ),
)


def _fin1_prep2_body(numa_ref, numb_ref, b1_ref, g1_ref, be1_ref,
                     w2_ref, as2_ref, ad2_ref,
                     h_out, inum_out, as_out, ad_out, maxs_out):
    num = jnp.concatenate([numa_ref[...][:N, :H // 2],
                           numb_ref[...][:N, :H // 2]], axis=1)
    den = numa_ref[...][:N, H // 2:H // 2 + 1]
    o = num / (den + 1e-16) + b1_ref[...].reshape(1, H)
    mu = jnp.mean(o, axis=0, keepdims=True)
    var = jnp.mean((o - mu) * (o - mu), axis=0, keepdims=True)
    o = (o - mu) / jnp.sqrt(var + 1e-5) * g1_ref[...].reshape(1, H) \
        + be1_ref[...].reshape(1, H)
    o = jnp.maximum(o, 0.0)
    h2 = jnp.dot(o, w2_ref[...], preferred_element_type=jnp.float32)
    a_s = jnp.dot(h2, as2_ref[...].reshape(H2, 1))
    a_d = jnp.dot(h2, ad2_ref[...].reshape(H2, 1))
    maxs = jnp.max(a_s)
    c = _lrelu(maxs + a_d)
    wself = jnp.exp(_lrelu(a_s + a_d) - c)
    h2f = _fold(h2)
    h_out[0] = h2f
    h_out[1] = h2f
    num0f = _fold_init(wself * h2, wself)
    inum_out[0] = num0f
    inum_out[1] = jnp.zeros((NP, DW), jnp.float32)
    as_out[...] = _pad_col(a_s)
    ad_out[...] = _pad_col(a_d)
    maxs_out[...] = jnp.full((1, LANES), maxs, jnp.float32)


_fin1_prep2 = pl.pallas_call(
    _fin1_prep2_body,
    out_shape=(
        jax.ShapeDtypeStruct((2, NP, DW), jnp.float32),
        jax.ShapeDtypeStruct((2, NP, DW), jnp.float32),
        jax.ShapeDtypeStruct((NP, 1), jnp.float32),
        jax.ShapeDtypeStruct((NP, 1), jnp.float32),
        jax.ShapeDtypeStruct((1, LANES), jnp.float32),
    ),
)


def _fin2_body(numa_ref, numb_ref, b2_ref, g2_ref, be2_ref, batch_ref,
               wfc_ref, bfc_ref, out_ref):
    acc = numa_ref[...][:N] + numb_ref[...][:N]
    num = acc[:, :H2]
    den = acc[:, H2:H2 + 1]
    o = num / (den + 1e-16) + b2_ref[...].reshape(1, H2)
    mu = jnp.mean(o, axis=0, keepdims=True)
    var = jnp.mean((o - mu) * (o - mu), axis=0, keepdims=True)
    o = (o - mu) / jnp.sqrt(var + 1e-5) * g2_ref[...].reshape(1, H2) \
        + be2_ref[...].reshape(1, H2)
    o = jnp.maximum(o, 0.0)
    grp = lax.broadcasted_iota(jnp.int32, (N, G), 1)
    P = (batch_ref[...] == grp).astype(jnp.float32)
    cnum = ((0,), (0,)), ((), ())
    pooled = lax.dot_general(P, o, dimension_numbers=cnum,
                             preferred_element_type=jnp.float32)
    counts = lax.dot_general(P, jnp.ones((N, 1), jnp.float32),
                             dimension_numbers=cnum,
                             preferred_element_type=jnp.float32)
    pooled = pooled / jnp.maximum(counts, 1.0)
    out_ref[...] = jnp.dot(pooled, wfc_ref[...],
                           preferred_element_type=jnp.float32) \
        + bfc_ref[...].reshape(1, OUT)


_fin2 = pl.pallas_call(
    _fin2_body,
    out_shape=jax.ShapeDtypeStruct((G, OUT), jnp.float32),
)


@functools.lru_cache(maxsize=None)
def _make_edge_kernel(edge_split):
    edges_per_tile = E // (NS * NC) if edge_split else E // NS
    nchunks = edges_per_tile // CH
    rpt = NP // NS

    mesh = plsc.VectorSubcoreMesh(core_axis_name="c", subcore_axis_name="s",
                                  num_cores=NC, num_subcores=NS)

    @functools.partial(
        pl.kernel,
        out_type=jax.ShapeDtypeStruct((NC, NP, DW), jnp.float32),
        mesh=mesh,
        compiler_params=pltpu.CompilerParams(use_tc_tiling_on_sc=False),
        scratch_types=dict(
            sh_num=pltpu.VMEM_SHARED((NP, DW), jnp.float32),
            sh_as=pltpu.VMEM_SHARED((NP,), jnp.float32),
            sh_ad=pltpu.VMEM_SHARED((NP,), jnp.float32),
            sidx=pltpu.VMEM((NSUB, SUB), jnp.int32),
            gidx=pltpu.VMEM((NSUB, SUB), jnp.int32),
            didx=pltpu.VMEM((NSUB, SUB), jnp.int32),
            asb=pltpu.VMEM((NSUB, SUB), jnp.float32),
            adb=pltpu.VMEM((NSUB, SUB), jnp.float32),
            wlin=pltpu.VMEM((CH,), jnp.float32),
            maxs_t=pltpu.VMEM((LANES,), jnp.float32),
            rows=pltpu.VMEM((NSUB, SUB, DW), jnp.float32),
            sem_s=pltpu.SemaphoreType.DMA,
            sem_r=pltpu.SemaphoreType.DMA,
        ),
    )
    def edge_kernel(srcg, dst3d, h_hbm, as_hbm, ad_hbm, maxs_hbm,
                    inum_hbm, num_out,
                    sh_num, sh_as, sh_ad, sidx, gidx, didx, asb, adb,
                    wlin, maxs_t, rows, sem_s, sem_r):
        cid = lax.axis_index("c")
        sid = lax.axis_index("s")
        r0 = sid * rpt

        pltpu.sync_copy(inum_hbm.at[cid, pl.ds(r0, rpt)],
                        sh_num.at[pl.ds(r0, rpt)])
        pltpu.sync_copy(as_hbm.at[pl.ds(r0, rpt)], sh_as.at[pl.ds(r0, rpt)])
        pltpu.sync_copy(ad_hbm.at[pl.ds(r0, rpt)], sh_ad.at[pl.ds(r0, rpt)])
        pltpu.sync_copy(maxs_hbm, maxs_t)
        plsc.subcore_barrier()

        maxv = maxs_t[...]
        if edge_split:
            chunk0 = (cid * NS + sid) * nchunks
        else:
            chunk0 = sid * nchunks

        def chunk_body(g, _):
            ci = chunk0 + g
            pltpu.sync_copy(srcg.at[0, ci], sidx)
            pltpu.sync_copy(srcg.at[cid, ci], gidx)
            pltpu.sync_copy(dst3d.at[ci], didx)

            cps = [pltpu.async_copy(sh_as.at[sidx.at[j]], asb.at[j], sem_s)
                   for j in range(NSUB)]
            cps += [pltpu.async_copy(sh_ad.at[didx.at[j]], adb.at[j], sem_s)
                    for j in range(NSUB)]
            rcps = [pltpu.async_copy(h_hbm.at[gidx.at[j]], rows.at[j], sem_r)
                    for j in range(NSUB)]
            for cp in cps:
                cp.wait()

            for i in range(CH // LANES):
                r, off = i // (SUB // LANES), (i % (SUB // LANES)) * LANES
                asg = asb[r, pl.ds(off, LANES)]
                adg = adb[r, pl.ds(off, LANES)]
                e = asg + adg
                e = jnp.where(e > 0, e, 0.2 * e)
                cg = maxv + adg
                cg = jnp.where(cg > 0, cg, 0.2 * cg)
                w = jnp.exp(e - cg)
                wlin[pl.ds(i * LANES, LANES)] = w

            for cp in rcps:
                cp.wait()

            for j in range(NSUB):
                def scale_body(m, _):
                    wv = wlin[pl.ds(j * SUB + m * LANES, LANES)]
                    for l in range(LANES):
                        k = m * LANES + l
                        wsc = jnp.full((LANES,), wv[l], jnp.float32)
                        for q in range(DW // LANES):
                            sl = pl.ds(q * LANES, LANES)
                            rows[j, k, sl] = rows[j, k, sl] * wsc
                    return 0

                lax.fori_loop(0, SUB // LANES, scale_body, 0)

            cps = [pltpu.async_copy(rows.at[j], sh_num.at[didx.at[j]], sem_r,
                                    add=True)
                   for j in range(NSUB)]
            for cp in cps:
                cp.wait()
            return 0

        lax.fori_loop(0, nchunks, chunk_body, 0)
        plsc.subcore_barrier()

        pltpu.sync_copy(sh_num.at[pl.ds(r0, rpt)],
                        num_out.at[cid, pl.ds(r0, rpt)])

    return edge_kernel


def kernel(x, edge_index, batch, W1, att_src1, att_dst1, b1, g1, be1,
           W2, att_src2, att_dst2, b2, g2, be2, Wfc, bfc):
    src3d = edge_index[0].reshape(E // CH, NSUB, SUB)
    dst3d = edge_index[1].reshape(E // CH, NSUB, SUB)
    srcg = jnp.stack([src3d, src3d + NP])

    _edge1 = _make_edge_kernel(False)
    _edge2 = _make_edge_kernel(True)

    h1, inum1, as1, ad1, maxs1 = _prep1(x, W1, att_src1, att_dst1)
    num1 = _edge1(srcg, dst3d, h1.reshape(2 * NP, DW),
                 as1.reshape(NP), ad1.reshape(NP), maxs1.reshape(LANES),
                 inum1)
    h2, inum2, as2, ad2, maxs2 = _fin1_prep2(
        num1[0], num1[1], b1, g1, be1, W2, att_src2, att_dst2)
    num2 = _edge2(srcg, dst3d, h2.reshape(2 * NP, DW),
                 as2.reshape(NP), ad2.reshape(NP), maxs2.reshape(LANES),
                 inum2)
    out = _fin2(num2[0], num2[1], b2, g2, be2, batch.reshape(N, 1), Wfc,
                bfc)
    return out
